# baseline (speedup 1.0000x reference)
import functools

import jax
import jax.numpy as jnp
from jax import lax
from jax.experimental import pallas as pl
from jax.experimental.pallas import tpu as pltpu
from jax.experimental.pallas import tpu_sc as plsc

N, E, NE, NQ, BETA = 16384, 256, 8192, 4, 0.25

BM = 256
CT = 512
NT = NE // CT

NC, NS = 2, 16
NW = NC * NS
ROWS_PER_W = N // NW
CH = 128
NCH = ROWS_PER_W // CH
LANES = 16
KV = E // LANES


_CHUNKS = (0, 2736, 5472, 8192)
_STAGE_CHUNKS = (_CHUNKS, _CHUNKS, _CHUNKS, _CHUNKS)

def _make_argmin_kernel(bounds):
    nchunks = len(bounds) - 1

    def kern(r_ref, r2_ref, cb_ref, c2_ref, idx_ref):
        r = r_ref[...]
        r2 = r2_ref[...]
        best = [None] * nchunks
        for t in range(NT):
            lo, hi = t * CT, (t + 1) * CT
            cb_t = cb_ref[lo:hi, :]
            mm = lax.dot_general(r, cb_t, (((1,), (1,)), ((), ())),
                                 preferred_element_type=jnp.float32)
            a = r2 + c2_ref[:, lo:hi]
            d = a - 2.0 * mm
            gcol = lax.broadcasted_iota(jnp.int32, (BM, CT), 1) + lo
            for k in range(nchunks):
                c0, c1 = bounds[k], bounds[k + 1]
                if c1 <= lo or c0 >= hi:
                    continue
                if c0 <= lo and c1 >= hi:
                    dm = d
                else:
                    seg = (gcol >= c0) & (gcol < c1)
                    dm = jnp.where(seg, d, jnp.float32(jnp.inf))
                tmin = jnp.min(dm, axis=1, keepdims=True)
                targ = jnp.min(jnp.where(dm == tmin, gcol, 2**30),
                               axis=1, keepdims=True)
                if best[k] is None:
                    best[k] = (tmin, targ)
                else:
                    bv, bi = best[k]
                    upd = tmin < bv
                    best[k] = (jnp.where(upd, tmin, bv),
                               jnp.where(upd, targ, bi))
        accv = best[0][0].astype(jnp.bfloat16).astype(jnp.float32)
        acci = best[0][1]
        for k in range(1, nchunks):
            mv, mi = best[k]
            take = mv < accv
            accv = jnp.where(take,
                             mv.astype(jnp.bfloat16).astype(jnp.float32),
                             accv)
            acci = jnp.where(take, mi, acci)
        idx_ref[...] = acci

    return kern


_ARGMIN_KERNELS = tuple(_make_argmin_kernel(b) for b in _STAGE_CHUNKS)


def _argmin_stage(q, r, r2, cb, c2):
    return pl.pallas_call(
        _ARGMIN_KERNELS[q],
        grid=(N // BM,),
        in_specs=[
            pl.BlockSpec((BM, E), lambda i: (i, 0)),
            pl.BlockSpec((BM, 1), lambda i: (i, 0)),
            pl.BlockSpec((NE, E), lambda i: (0, 0)),
            pl.BlockSpec((1, NE), lambda i: (0, 0)),
        ],
        out_specs=pl.BlockSpec((BM, 1), lambda i: (i, 0)),
        out_shape=jax.ShapeDtypeStruct((N, 1), jnp.int32),
        compiler_params=pltpu.CompilerParams(
            dimension_semantics=("arbitrary",)),
    )(r, r2, cb, c2)


_SC_MESH = plsc.VectorSubcoreMesh(
    core_axis_name="c", subcore_axis_name="s", num_cores=NC, num_subcores=NS)


@functools.partial(
    pl.kernel,
    out_type=(
        jax.ShapeDtypeStruct((N, E), jnp.float32),
        jax.ShapeDtypeStruct((NW, LANES), jnp.float32),
    ),
    mesh=_SC_MESH,
    scratch_types=[
        pltpu.VMEM((CH,), jnp.int32),
        pltpu.VMEM((CH, E), jnp.float32),
        pltpu.VMEM((CH, E), jnp.float32),
        pltpu.VMEM((LANES,), jnp.float32),
        pltpu.SemaphoreType.DMA,
    ],
)
def _sc_gather_update(idx_hbm, cb_hbm, r_hbm, rnew_hbm, part_hbm,
                      idx_v, g_v, r_v, acc_v, sem):
    wid = lax.axis_index("s") * NC + lax.axis_index("c")
    acc = jnp.zeros((LANES,), jnp.float32)
    for c in range(NCH):
        base = wid * ROWS_PER_W + c * CH
        pltpu.sync_copy(idx_hbm.at[pl.ds(base, CH)], idx_v)
        pltpu.async_copy(cb_hbm.at[idx_v], g_v, sem).wait()
        pltpu.sync_copy(r_hbm.at[pl.ds(base, CH)], r_v)

        def body(i, acc):
            for k in range(KV):
                g16 = g_v[i, pl.ds(k * LANES, LANES)]
                r16 = r_v[i, pl.ds(k * LANES, LANES)]
                t16 = g16 - r16
                xres = r16 + t16
                r_v[i, pl.ds(k * LANES, LANES)] = r16 - xres
                acc = acc + t16 * t16
            return acc

        acc = lax.fori_loop(0, CH, body, acc)
        pltpu.sync_copy(r_v, rnew_hbm.at[pl.ds(base, CH)])
    acc_v[...] = acc
    pltpu.sync_copy(acc_v, part_hbm.at[wid])


def kernel(x, codebooks):
    residual = x
    losses = []
    indices = []
    for q in range(NQ):
        cb = codebooks[q]
        r2 = jnp.sum(residual ** 2, axis=1, keepdims=True)
        c2 = jnp.sum(cb ** 2, axis=1)[None, :]
        idx2d = _argmin_stage(q, residual, r2, cb, c2)
        idx = idx2d.reshape(N)
        rnew, part = _sc_gather_update(idx, cb, residual)
        L = part.sum() / (N * E)
        losses.append(L + BETA * L)
        indices.append(idx)
        residual = rnew
    x_q = x - residual
    mean_losses = jnp.stack(losses).mean()
    all_indices = jnp.stack(indices, axis=-1)
    return (x_q, mean_losses, all_indices)


# speedup vs baseline: 1.3385x; 1.3385x over previous
import functools

import jax
import jax.numpy as jnp
from jax import lax
from jax.experimental import pallas as pl
from jax.experimental.pallas import tpu as pltpu
from jax.experimental.pallas import tpu_sc as plsc

N, E, NE, NQ, BETA = 16384, 256, 8192, 4, 0.25

BM = 256
CT = 512
NT = NE // CT

NC, NS = 2, 16
NW = NC * NS
ROWS_PER_W = N // NW
CH = 128
NCH = ROWS_PER_W // CH
LANES = 16
KV = E // LANES


_CHUNKS = (0, 2736, 5472, 8192)
_STAGE_CHUNKS = (_CHUNKS, _CHUNKS, _CHUNKS, _CHUNKS)

def _make_argmin_kernel(bounds):
    nchunks = len(bounds) - 1

    def kern(r_ref, r2_ref, cb_ref, c2_ref, idx_ref):
        r = r_ref[...]
        r2 = r2_ref[...]
        vacc = [None] * nchunks
        tacc = [None] * nchunks
        for t in range(NT):
            lo, hi = t * CT, (t + 1) * CT
            cb_t = cb_ref[lo:hi, :]
            mm = lax.dot_general(r, cb_t, (((1,), (1,)), ((), ())),
                                 preferred_element_type=jnp.float32)
            a = r2 + c2_ref[:, lo:hi]
            d = a - 2.0 * mm
            for k in range(nchunks):
                c0, c1 = bounds[k], bounds[k + 1]
                if c1 <= lo or c0 >= hi:
                    continue
                if c0 <= lo and c1 >= hi:
                    dm = d
                else:
                    gcol = lax.broadcasted_iota(jnp.int32, (BM, CT), 1) + lo
                    seg = (gcol >= c0) & (gcol < c1)
                    dm = jnp.where(seg, d, jnp.float32(jnp.inf))
                if vacc[k] is None:
                    vacc[k] = dm
                    tacc[k] = jnp.full((BM, CT), t, jnp.int32)
                else:
                    lt = dm < vacc[k]
                    vacc[k] = jnp.minimum(vacc[k], dm)
                    tacc[k] = jnp.where(lt, jnp.int32(t), tacc[k])
        lane = lax.broadcasted_iota(jnp.int32, (BM, CT), 1)
        best = []
        for k in range(nchunks):
            tmin = jnp.min(vacc[k], axis=1, keepdims=True)
            gidx = tacc[k].astype(jnp.int32) * CT + lane
            targ = jnp.min(jnp.where(vacc[k] == tmin, gidx, 2**30),
                           axis=1, keepdims=True)
            best.append((tmin, targ))
        accv = best[0][0].astype(jnp.bfloat16).astype(jnp.float32)
        acci = best[0][1]
        for k in range(1, nchunks):
            mv, mi = best[k]
            take = mv < accv
            accv = jnp.where(take,
                             mv.astype(jnp.bfloat16).astype(jnp.float32),
                             accv)
            acci = jnp.where(take, mi, acci)
        idx_ref[...] = acci

    return kern


_ARGMIN_KERNELS = tuple(_make_argmin_kernel(b) for b in _STAGE_CHUNKS)


def _argmin_stage(q, r, r2, cb, c2):
    return pl.pallas_call(
        _ARGMIN_KERNELS[q],
        grid=(N // BM,),
        in_specs=[
            pl.BlockSpec((BM, E), lambda i: (i, 0)),
            pl.BlockSpec((BM, 1), lambda i: (i, 0)),
            pl.BlockSpec((NE, E), lambda i: (0, 0)),
            pl.BlockSpec((1, NE), lambda i: (0, 0)),
        ],
        out_specs=pl.BlockSpec((BM, 1), lambda i: (i, 0)),
        out_shape=jax.ShapeDtypeStruct((N, 1), jnp.int32),
        compiler_params=pltpu.CompilerParams(
            dimension_semantics=("arbitrary",)),
    )(r, r2, cb, c2)


_SC_MESH = plsc.VectorSubcoreMesh(
    core_axis_name="c", subcore_axis_name="s", num_cores=NC, num_subcores=NS)


@functools.partial(
    pl.kernel,
    out_type=(
        jax.ShapeDtypeStruct((N, E), jnp.float32),
        jax.ShapeDtypeStruct((NW, LANES), jnp.float32),
    ),
    mesh=_SC_MESH,
    scratch_types=[
        pltpu.VMEM((CH,), jnp.int32),
        pltpu.VMEM((CH, E), jnp.float32),
        pltpu.VMEM((CH, E), jnp.float32),
        pltpu.VMEM((LANES,), jnp.float32),
        pltpu.SemaphoreType.DMA,
    ],
)
def _sc_gather_update(idx_hbm, cb_hbm, r_hbm, rnew_hbm, part_hbm,
                      idx_v, g_v, r_v, acc_v, sem):
    wid = lax.axis_index("s") * NC + lax.axis_index("c")
    acc = jnp.zeros((LANES,), jnp.float32)
    for c in range(NCH):
        base = wid * ROWS_PER_W + c * CH
        pltpu.sync_copy(idx_hbm.at[pl.ds(base, CH)], idx_v)
        pltpu.async_copy(cb_hbm.at[idx_v], g_v, sem).wait()
        pltpu.sync_copy(r_hbm.at[pl.ds(base, CH)], r_v)

        def body(i, acc):
            for k in range(KV):
                g16 = g_v[i, pl.ds(k * LANES, LANES)]
                r16 = r_v[i, pl.ds(k * LANES, LANES)]
                t16 = g16 - r16
                xres = r16 + t16
                r_v[i, pl.ds(k * LANES, LANES)] = r16 - xres
                acc = acc + t16 * t16
            return acc

        acc = lax.fori_loop(0, CH, body, acc)
        pltpu.sync_copy(r_v, rnew_hbm.at[pl.ds(base, CH)])
    acc_v[...] = acc
    pltpu.sync_copy(acc_v, part_hbm.at[wid])


def kernel(x, codebooks):
    residual = x
    losses = []
    indices = []
    for q in range(NQ):
        cb = codebooks[q]
        r2 = jnp.sum(residual ** 2, axis=1, keepdims=True)
        c2 = jnp.sum(cb ** 2, axis=1)[None, :]
        idx2d = _argmin_stage(q, residual, r2, cb, c2)
        idx = idx2d.reshape(N)
        rnew, part = _sc_gather_update(idx, cb, residual)
        L = part.sum() / (N * E)
        losses.append(L + BETA * L)
        indices.append(idx)
        residual = rnew
    x_q = x - residual
    mean_losses = jnp.stack(losses).mean()
    all_indices = jnp.stack(indices, axis=-1)
    return (x_q, mean_losses, all_indices)


# speedup vs baseline: 1.4548x; 1.0869x over previous
import functools

import jax
import jax.numpy as jnp
from jax import lax
from jax.experimental import pallas as pl
from jax.experimental.pallas import tpu as pltpu
from jax.experimental.pallas import tpu_sc as plsc

N, E, NE, NQ, BETA = 16384, 256, 8192, 4, 0.25

BM = 512
CT = 512
NT = NE // CT

NC, NS = 2, 16
NW = NC * NS
ROWS_PER_W = N // NW
CH = 128
NCH = ROWS_PER_W // CH
LANES = 16
KV = E // LANES


_CHUNKS = (0, 2736, 5472, 8192)
_STAGE_CHUNKS = (_CHUNKS, _CHUNKS, _CHUNKS, _CHUNKS)

def _make_argmin_kernel(bounds):
    nchunks = len(bounds) - 1

    def kern(r_ref, r2_ref, cb_ref, c2_ref, idx_ref):
        r = r_ref[...]
        r2 = r2_ref[...]
        vacc = [None] * nchunks
        tacc = [None] * nchunks
        for t in range(NT):
            lo, hi = t * CT, (t + 1) * CT
            cb_t = cb_ref[lo:hi, :]
            mm = lax.dot_general(r, cb_t, (((1,), (1,)), ((), ())),
                                 preferred_element_type=jnp.float32)
            a = r2 + c2_ref[:, lo:hi]
            d = a - 2.0 * mm
            for k in range(nchunks):
                c0, c1 = bounds[k], bounds[k + 1]
                if c1 <= lo or c0 >= hi:
                    continue
                if c0 <= lo and c1 >= hi:
                    dm = d
                else:
                    gcol = lax.broadcasted_iota(jnp.int32, (BM, CT), 1) + lo
                    seg = (gcol >= c0) & (gcol < c1)
                    dm = jnp.where(seg, d, jnp.float32(jnp.inf))
                if vacc[k] is None:
                    vacc[k] = dm
                    tacc[k] = jnp.full((BM, CT), t, jnp.int32)
                else:
                    lt = dm < vacc[k]
                    vacc[k] = jnp.minimum(vacc[k], dm)
                    tacc[k] = jnp.where(lt, jnp.int32(t), tacc[k])
        lane = lax.broadcasted_iota(jnp.int32, (BM, CT), 1)
        best = []
        for k in range(nchunks):
            tmin = jnp.min(vacc[k], axis=1, keepdims=True)
            gidx = tacc[k].astype(jnp.int32) * CT + lane
            targ = jnp.min(jnp.where(vacc[k] == tmin, gidx, 2**30),
                           axis=1, keepdims=True)
            best.append((tmin, targ))
        accv = best[0][0].astype(jnp.bfloat16).astype(jnp.float32)
        acci = best[0][1]
        for k in range(1, nchunks):
            mv, mi = best[k]
            take = mv < accv
            accv = jnp.where(take,
                             mv.astype(jnp.bfloat16).astype(jnp.float32),
                             accv)
            acci = jnp.where(take, mi, acci)
        idx_ref[...] = acci

    return kern


_ARGMIN_KERNELS = tuple(_make_argmin_kernel(b) for b in _STAGE_CHUNKS)


def _argmin_stage(q, r, r2, cb, c2):
    return pl.pallas_call(
        _ARGMIN_KERNELS[q],
        grid=(N // BM,),
        in_specs=[
            pl.BlockSpec((BM, E), lambda i: (i, 0)),
            pl.BlockSpec((BM, 1), lambda i: (i, 0)),
            pl.BlockSpec((NE, E), lambda i: (0, 0)),
            pl.BlockSpec((1, NE), lambda i: (0, 0)),
        ],
        out_specs=pl.BlockSpec((BM, 1), lambda i: (i, 0)),
        out_shape=jax.ShapeDtypeStruct((N, 1), jnp.int32),
        compiler_params=pltpu.CompilerParams(
            dimension_semantics=("arbitrary",)),
    )(r, r2, cb, c2)


_SC_MESH = plsc.VectorSubcoreMesh(
    core_axis_name="c", subcore_axis_name="s", num_cores=NC, num_subcores=NS)


@functools.partial(
    pl.kernel,
    out_type=(
        jax.ShapeDtypeStruct((N, E), jnp.float32),
        jax.ShapeDtypeStruct((NW, LANES), jnp.float32),
    ),
    mesh=_SC_MESH,
    scratch_types=[
        pltpu.VMEM((CH,), jnp.int32),
        pltpu.VMEM((CH, E), jnp.float32),
        pltpu.VMEM((CH, E), jnp.float32),
        pltpu.VMEM((LANES,), jnp.float32),
        pltpu.SemaphoreType.DMA,
    ],
)
def _sc_gather_update(idx_hbm, cb_hbm, r_hbm, rnew_hbm, part_hbm,
                      idx_v, g_v, r_v, acc_v, sem):
    wid = lax.axis_index("s") * NC + lax.axis_index("c")
    acc = jnp.zeros((LANES,), jnp.float32)
    for c in range(NCH):
        base = wid * ROWS_PER_W + c * CH
        pltpu.sync_copy(idx_hbm.at[pl.ds(base, CH)], idx_v)
        pltpu.async_copy(cb_hbm.at[idx_v], g_v, sem).wait()
        pltpu.sync_copy(r_hbm.at[pl.ds(base, CH)], r_v)

        def body(i, acc):
            for k in range(KV):
                g16 = g_v[i, pl.ds(k * LANES, LANES)]
                r16 = r_v[i, pl.ds(k * LANES, LANES)]
                t16 = g16 - r16
                xres = r16 + t16
                r_v[i, pl.ds(k * LANES, LANES)] = r16 - xres
                acc = acc + t16 * t16
            return acc

        acc = lax.fori_loop(0, CH, body, acc)
        pltpu.sync_copy(r_v, rnew_hbm.at[pl.ds(base, CH)])
    acc_v[...] = acc
    pltpu.sync_copy(acc_v, part_hbm.at[wid])


def kernel(x, codebooks):
    residual = x
    losses = []
    indices = []
    for q in range(NQ):
        cb = codebooks[q]
        r2 = jnp.sum(residual ** 2, axis=1, keepdims=True)
        c2 = jnp.sum(cb ** 2, axis=1)[None, :]
        idx2d = _argmin_stage(q, residual, r2, cb, c2)
        idx = idx2d.reshape(N)
        rnew, part = _sc_gather_update(idx, cb, residual)
        L = part.sum() / (N * E)
        losses.append(L + BETA * L)
        indices.append(idx)
        residual = rnew
    x_q = x - residual
    mean_losses = jnp.stack(losses).mean()
    all_indices = jnp.stack(indices, axis=-1)
    return (x_q, mean_losses, all_indices)
